# Initial kernel scaffold; baseline (speedup 1.0000x reference)
import jax
import jax.numpy as jnp
from jax.experimental import pallas as pl


def kernel(inputs, W1_shared, W2_shared, W1_routed, W2_routed, centroids, expert_biases):
    raise NotImplementedError("write your pallas kernel here")


# speedup vs baseline: 1.3306x; 1.3306x over previous
import functools

import jax
import jax.numpy as jnp
from jax import lax
from jax.experimental import pallas as pl
from jax.experimental.pallas import tpu as pltpu
from jax.experimental.pallas import tpu_sc as plsc

D_MODEL_C = 1024
D_FF_C = 2048
E_C = 8
T_C = 2048
P_C = 2 * T_C
BLK = 128
NBLK = P_C // BLK + E_C
ROWS = NBLK * BLK
CHUNK = 256


def _router_body(x_ref, ct_ref, b_ref, slot_ref, w0_ref, w1_ref, be_ref):
    aff = jnp.dot(x_ref[...], ct_ref[...], preferred_element_type=jnp.float32)
    biased = aff + b_ref[...]
    iota_e = lax.broadcasted_iota(jnp.int32, (T_C, E_C), 1)
    m1 = jnp.max(biased, axis=1, keepdims=True)
    idx1 = jnp.min(jnp.where(biased == m1, iota_e, E_C), axis=1, keepdims=True)
    sel1 = iota_e == idx1
    masked = jnp.where(sel1, -1e30, biased)
    m2 = jnp.max(masked, axis=1, keepdims=True)
    idx2 = jnp.min(jnp.where(masked == m2, iota_e, E_C), axis=1, keepdims=True)
    sel2 = iota_e == idx2
    s1 = jnp.sum(jnp.where(sel1, aff, 0.0), axis=1, keepdims=True)
    s2 = jnp.sum(jnp.where(sel2, aff, 0.0), axis=1, keepdims=True)
    g1 = 1.0 / (1.0 + jnp.exp(-s1))
    g2 = 1.0 / (1.0 + jnp.exp(-s2))
    den = g1 + g2 + 1e-8
    w0_ref[...] = jnp.broadcast_to(g1 / den, (T_C, 16))
    w1_ref[...] = jnp.broadcast_to(g2 / den, (T_C, 16))

    P = jnp.concatenate([sel1.astype(jnp.float32), sel2.astype(jnp.float32)], axis=0)

    nch = P_C // CHUNK
    totals = jnp.concatenate(
        [jnp.sum(P[c * CHUNK:(c + 1) * CHUNK], axis=0, keepdims=True) for c in range(nch)],
        axis=0)
    rC = lax.broadcasted_iota(jnp.int32, (nch, nch), 0)
    cC = lax.broadcasted_iota(jnp.int32, (nch, nch), 1)
    Lc = (rC > cC).astype(jnp.float32)
    chunk_off = jnp.dot(Lc, totals, preferred_element_type=jnp.float32)
    counts = jnp.sum(totals, axis=0, keepdims=True)
    nb = (counts.astype(jnp.int32) + (BLK - 1)) // BLK
    aligned = (nb * BLK).astype(jnp.float32)

    rE = lax.broadcasted_iota(jnp.int32, (E_C, E_C), 0)
    cE = lax.broadcasted_iota(jnp.int32, (E_C, E_C), 1)
    Mgt = (cE < rE).astype(jnp.float32)
    r2 = lax.broadcasted_iota(jnp.int32, (CHUNK, CHUNK), 0)
    c2 = lax.broadcasted_iota(jnp.int32, (CHUNK, CHUNK), 1)
    Ls = (r2 > c2).astype(jnp.float32)

    for c in range(nch):
        Pc = P[c * CHUNK:(c + 1) * CHUNK]
        rank_c = jnp.dot(Ls, Pc, preferred_element_type=jnp.float32) + chunk_off[c:c + 1, :]
        G = jnp.dot(Pc, Mgt, preferred_element_type=jnp.float32)
        offg = jnp.sum(G * aligned, axis=1, keepdims=True)
        rnk = jnp.sum(Pc * rank_c, axis=1, keepdims=True)
        slot_ref[c * CHUNK:(c + 1) * CHUNK, :] = (rnk + offg).astype(jnp.int32)

    off8 = jnp.sum(jnp.broadcast_to(aligned, (E_C, E_C)) * Mgt, axis=1, keepdims=True)
    bvals = (lax.broadcasted_iota(jnp.int32, (E_C, NBLK), 1) * BLK).astype(jnp.float32)
    cmp = (off8 <= bvals).astype(jnp.int32)
    be_ref[...] = jnp.sum(cmp, axis=0, keepdims=True) - 1


def _router(x, cT, b_row):
    return pl.pallas_call(
        _router_body,
        out_shape=[
            jax.ShapeDtypeStruct((P_C, 1), jnp.int32),
            jax.ShapeDtypeStruct((T_C, 16), jnp.float32),
            jax.ShapeDtypeStruct((T_C, 16), jnp.float32),
            jax.ShapeDtypeStruct((1, NBLK), jnp.int32),
        ],
    )(x, cT, b_row)


def _ffn_body(x_ref, w1_ref, w2_ref, o_ref):
    h = jnp.dot(x_ref[...], w1_ref[...], preferred_element_type=jnp.float32)
    h = h * (1.0 / (1.0 + jnp.exp(-h)))
    o_ref[...] = jnp.dot(h, w2_ref[...], preferred_element_type=jnp.float32)


def _shared_ffn(x, W1, W2):
    return pl.pallas_call(
        _ffn_body,
        grid=(T_C // BLK,),
        in_specs=[
            pl.BlockSpec((BLK, D_MODEL_C), lambda b: (b, 0)),
            pl.BlockSpec((D_MODEL_C, D_FF_C), lambda b: (0, 0)),
            pl.BlockSpec((D_FF_C, D_MODEL_C), lambda b: (0, 0)),
        ],
        out_specs=pl.BlockSpec((BLK, D_MODEL_C), lambda b: (b, 0)),
        out_shape=jax.ShapeDtypeStruct((T_C, D_MODEL_C), jnp.float32),
    )(x, W1, W2)


def _grouped_body(be_ref, xs_ref, w1_ref, w2_ref, ys_ref):
    h = jnp.dot(xs_ref[...], w1_ref[0], preferred_element_type=jnp.float32)
    h = h * (1.0 / (1.0 + jnp.exp(-h)))
    ys_ref[...] = jnp.dot(h, w2_ref[0], preferred_element_type=jnp.float32)


def _grouped_ffn(be, xs, W1r, W2r):
    grid_spec = pltpu.PrefetchScalarGridSpec(
        num_scalar_prefetch=1,
        grid=(NBLK,),
        in_specs=[
            pl.BlockSpec((BLK, D_MODEL_C), lambda b, be: (b, 0)),
            pl.BlockSpec((1, D_MODEL_C, D_FF_C), lambda b, be: (be[b], 0, 0)),
            pl.BlockSpec((1, D_FF_C, D_MODEL_C), lambda b, be: (be[b], 0, 0)),
        ],
        out_specs=pl.BlockSpec((BLK, D_MODEL_C), lambda b, be: (b, 0)),
    )
    return pl.pallas_call(
        _grouped_body,
        grid_spec=grid_spec,
        out_shape=jax.ShapeDtypeStruct((ROWS, D_MODEL_C), jnp.float32),
    )(be, xs, W1r, W2r)


_DCHUNK = 64
_CCHUNK = 16


def _dispatch_body(x_hbm, slot_hbm, xs_hbm, sl_v, rows_v, sem):
    wid = lax.axis_index("s") * 2 + lax.axis_index("c")
    k = wid // 16
    t_base = 128 * wid - T_C * k
    p_base = 128 * wid
    for c in range(128 // _DCHUNK):
        pltpu.sync_copy(slot_hbm.at[pl.ds(p_base + c * _DCHUNK, _DCHUNK)], sl_v)
        pltpu.sync_copy(x_hbm.at[pl.ds(t_base + c * _DCHUNK, _DCHUNK)], rows_v)
        pltpu.async_copy(rows_v, xs_hbm.at[sl_v], sem).wait()


def _combine_body(ys_hbm, s0_hbm, s1_hbm, w0_hbm, w1_hbm, sh_hbm, out_hbm,
                  s0_v, s1_v, g0_v, g1_v, acc_v, w0_v, w1_v, sem):
    wid = lax.axis_index("s") * 2 + lax.axis_index("c")
    for c in range(64 // _CCHUNK):
        tb = 64 * wid + c * _CCHUNK
        pltpu.sync_copy(s0_hbm.at[pl.ds(tb, _CCHUNK)], s0_v)
        pltpu.sync_copy(s1_hbm.at[pl.ds(tb, _CCHUNK)], s1_v)
        pltpu.async_copy(ys_hbm.at[s0_v], g0_v, sem).wait()
        pltpu.async_copy(ys_hbm.at[s1_v], g1_v, sem).wait()
        pltpu.sync_copy(sh_hbm.at[pl.ds(tb, _CCHUNK)], acc_v)
        pltpu.sync_copy(w0_hbm.at[pl.ds(tb, _CCHUNK)], w0_v)
        pltpu.sync_copy(w1_hbm.at[pl.ds(tb, _CCHUNK)], w1_v)

        def body_i(i, _):
            def body_j(j, _):
                seg = pl.ds(j * 16, 16)
                acc_v[i, seg] = (acc_v[i, seg]
                                 + w0_v[i, :] * g0_v[i, seg]
                                 + w1_v[i, :] * g1_v[i, seg])
                return 0
            lax.fori_loop(0, D_MODEL_C // 16, body_j, 0)
            return 0
        lax.fori_loop(0, _CCHUNK, body_i, 0)
        pltpu.sync_copy(acc_v, out_hbm.at[pl.ds(tb, _CCHUNK)])


@functools.cache
def _sc_kernels():
    mesh = plsc.VectorSubcoreMesh(core_axis_name="c", subcore_axis_name="s")
    dispatch = pl.kernel(
        _dispatch_body,
        mesh=mesh,
        out_type=jax.ShapeDtypeStruct((ROWS, D_MODEL_C), jnp.float32),
        scratch_types=[
            pltpu.VMEM((_DCHUNK,), jnp.int32),
            pltpu.VMEM((_DCHUNK, D_MODEL_C), jnp.float32),
            pltpu.SemaphoreType.DMA,
        ],
    )
    combine = pl.kernel(
        _combine_body,
        mesh=mesh,
        out_type=jax.ShapeDtypeStruct((T_C, D_MODEL_C), jnp.float32),
        scratch_types=[
            pltpu.VMEM((_CCHUNK,), jnp.int32),
            pltpu.VMEM((_CCHUNK,), jnp.int32),
            pltpu.VMEM((_CCHUNK, D_MODEL_C), jnp.float32),
            pltpu.VMEM((_CCHUNK, D_MODEL_C), jnp.float32),
            pltpu.VMEM((_CCHUNK, D_MODEL_C), jnp.float32),
            pltpu.VMEM((_CCHUNK, 16), jnp.float32),
            pltpu.VMEM((_CCHUNK, 16), jnp.float32),
            pltpu.SemaphoreType.DMA,
        ],
    )
    return dispatch, combine


def kernel(inputs, W1_shared, W2_shared, W1_routed, W2_routed, centroids, expert_biases):
    x = inputs.reshape(T_C, D_MODEL_C)
    cT = centroids.T
    b_row = expert_biases.reshape(1, E_C)

    slot2, w0b, w1b, be2 = _router(x, cT, b_row)
    slot = slot2.reshape(P_C)
    be = be2.reshape(NBLK)

    dispatch, combine = _sc_kernels()
    shared = _shared_ffn(x, W1_shared, W2_shared)
    xs = dispatch(x, slot)
    ys = _grouped_ffn(be, xs, W1_routed, W2_routed)
    out = combine(ys, slot[:T_C], slot[T_C:], w0b, w1b, shared)
    return out.reshape(1, T_C, D_MODEL_C)
